# baseline (speedup 1.0000x reference)
import functools

import jax
import jax.numpy as jnp
from jax import lax
from jax.experimental import pallas as pl
from jax.experimental.pallas import tpu as pltpu
from jax.experimental.pallas import tpu_sc as plsc

NC = 2
NS = 16
NW = NC * NS
CH = 128


def _sc_mesh():
  return plsc.VectorSubcoreMesh(
      core_axis_name="c", subcore_axis_name="s", num_cores=NC,
      num_subcores=NS)


_SC_PARAMS = pltpu.CompilerParams(use_tc_tiling_on_sc=False)


NB = 4
G = 2


def _sc_degree(dst3, ones_hbm, zeros_hbm, npad, nch):
  rpt = npad // NS

  @functools.partial(
      pl.kernel,
      mesh=_sc_mesh(),
      compiler_params=_SC_PARAMS,
      out_type=jax.ShapeDtypeStruct((NC, npad, 16), jnp.float32),
      scratch_types=[
          pltpu.VMEM((nch // G, G * CH), jnp.int32),
          pltpu.VMEM((G * CH, 16), jnp.float32),
          pltpu.VMEM_SHARED((npad, 16), jnp.float32),
      ],
  )
  def k(dst_hbm, ones_h, z_hbm, out_hbm, didx, ones_v, acc):
    c = lax.axis_index("c")
    s = lax.axis_index("s")
    wid = c * NS + s
    pltpu.sync_copy(ones_h, ones_v)
    pltpu.sync_copy(dst_hbm.at[wid], didx)
    pltpu.sync_copy(z_hbm.at[pl.ds(s * rpt, rpt)], acc.at[pl.ds(s * rpt, rpt)])
    plsc.subcore_barrier()

    @pl.loop(0, nch // G)
    def _(j):
      pltpu.sync_copy(ones_v, acc.at[didx.at[j]], add=True)

    plsc.subcore_barrier()
    pltpu.sync_copy(acc.at[pl.ds(s * rpt, rpt)],
                    out_hbm.at[c, pl.ds(s * rpt, rpt)])

  return k(dst3, ones_hbm, zeros_hbm)


def _sc_scatter(u, src3, dst3, zeros_hbm, npad, nch):
  d = u.shape[1]
  rpt = npad // NS

  @functools.partial(
      pl.kernel,
      mesh=_sc_mesh(),
      compiler_params=_SC_PARAMS,
      out_type=jax.ShapeDtypeStruct((NC, npad, d), jnp.float32),
      scratch_types=[
          pltpu.VMEM((nch // G, G * CH), jnp.int32),
          pltpu.VMEM((nch // G, G * CH), jnp.int32),
          pltpu.VMEM((G * CH, d), jnp.float32),
          pltpu.VMEM_SHARED((npad, d), jnp.float32),
          pltpu.SemaphoreType.DMA,
      ],
  )
  def k(u_hbm, src_hbm, dst_hbm, z_hbm, out_hbm, sidx, didx, rows, acc, sem):
    c = lax.axis_index("c")
    s = lax.axis_index("s")
    wid = c * NS + s
    pltpu.sync_copy(src_hbm.at[wid], sidx)
    pltpu.sync_copy(dst_hbm.at[wid], didx)
    pltpu.sync_copy(z_hbm.at[pl.ds(s * rpt, rpt)], acc.at[pl.ds(s * rpt, rpt)])
    plsc.subcore_barrier()

    @pl.loop(0, nch // G)
    def _(i):
      pltpu.async_copy(u_hbm.at[sidx.at[i]], rows, sem).wait()
      pltpu.sync_copy(rows, acc.at[didx.at[i]], add=True)

    plsc.subcore_barrier()
    pltpu.sync_copy(acc.at[pl.ds(s * rpt, rpt)],
                    out_hbm.at[c, pl.ds(s * rpt, rpt)])

  return k(u, src3, dst3, zeros_hbm)


def _tc_pass_a(degp, x, w1, n):
  dh = w1.shape[1]

  def body(degp_ref, x_ref, w_ref, u_ref, dinv_ref):
    deg = degp_ref[0] + degp_ref[1]
    dinv = lax.rsqrt(deg[:n, 0:1] + 1.0)
    h = jnp.dot(x_ref[...], w_ref[...], preferred_element_type=jnp.float32)
    u_ref[...] = h * dinv
    dinv_ref[...] = dinv

  return pl.pallas_call(
      body,
      out_shape=(
          jax.ShapeDtypeStruct((n, dh), jnp.float32),
          jax.ShapeDtypeStruct((n, 1), jnp.float32),
      ),
  )(degp, x, w1)


def _tc_pass_b(s1, u1, dinv, b1, w2, n):
  do = w2.shape[1]

  def body(s_ref, u_ref, dinv_ref, b_ref, w_ref, u2_ref):
    agg = (s_ref[0, :n] + s_ref[1, :n] + u_ref[...]) * dinv_ref[...]
    h1 = jnp.maximum(agg + b_ref[...], 0.0)
    h2 = jnp.dot(h1, w_ref[...], preferred_element_type=jnp.float32)
    u2_ref[...] = h2 * dinv_ref[...]

  return pl.pallas_call(
      body,
      out_shape=jax.ShapeDtypeStruct((n, do), jnp.float32),
  )(s1, u1, dinv, b1, w2)


def _tc_pass_c(s2, u2, dinv, b2, n):
  do = u2.shape[1]

  def body(s_ref, u_ref, dinv_ref, b_ref, out_ref):
    agg = (s_ref[0, :n] + s_ref[1, :n] + u_ref[...]) * dinv_ref[...]
    out_ref[...] = agg + b_ref[...]

  return pl.pallas_call(
      body,
      out_shape=jax.ShapeDtypeStruct((n, do), jnp.float32),
  )(s2, u2, dinv, b2)


def kernel(x, edge_index, W1, b1, W2, b2):
  n = x.shape[0]
  e = edge_index.shape[1]
  dh = W1.shape[1]
  do = W2.shape[1]

  nch = -(-e // (NW * CH * NB)) * NB
  epad = NW * nch * CH
  src = jnp.concatenate(
      [edge_index[0], jnp.zeros((epad - e,), edge_index.dtype)])
  dst = jnp.concatenate(
      [edge_index[1], jnp.full((epad - e,), n, edge_index.dtype)])
  dst3 = dst.reshape(NW, nch // G, G * CH)
  src3 = src.reshape(NW, nch // G, G * CH)

  npad = -(-(n + 1) // (NS * 8)) * (NS * 8)
  ones16 = jnp.ones((G * CH, 16), jnp.float32)
  z16 = jnp.zeros((npad, 16), jnp.float32)
  zh = jnp.zeros((npad, dh), jnp.float32)
  zo = jnp.zeros((npad, do), jnp.float32)

  degp = _sc_degree(dst3, ones16, z16, npad, nch)
  u1, dinv = _tc_pass_a(degp, x, W1, n)
  s1 = _sc_scatter(u1, src3, dst3, zh, npad, nch)
  u2 = _tc_pass_b(s1, u1, dinv, b1, W2, n)
  s2 = _sc_scatter(u2, src3, dst3, zo, npad, nch)
  return _tc_pass_c(s2, u2, dinv, b2, n)


# speedup vs baseline: 1.3524x; 1.3524x over previous
import functools

import jax
import jax.numpy as jnp
from jax import lax
from jax.experimental import pallas as pl
from jax.experimental.pallas import tpu as pltpu
from jax.experimental.pallas import tpu_sc as plsc

NC = 2
NS = 16
NW = NC * NS
CH = 128


def _sc_mesh():
  return plsc.VectorSubcoreMesh(
      core_axis_name="c", subcore_axis_name="s", num_cores=NC,
      num_subcores=NS)


_SC_PARAMS = pltpu.CompilerParams(use_tc_tiling_on_sc=False)


NB = 4
G = 4


def _sc_degree(dst3, ones_hbm, zeros_hbm, npad, nch):
  rpt = npad // NS

  @functools.partial(
      pl.kernel,
      mesh=_sc_mesh(),
      compiler_params=_SC_PARAMS,
      out_type=jax.ShapeDtypeStruct((NC, npad, 16), jnp.float32),
      scratch_types=[
          pltpu.VMEM((nch // G, G * CH), jnp.int32),
          pltpu.VMEM((G * CH, 16), jnp.float32),
          pltpu.VMEM_SHARED((npad, 16), jnp.float32),
      ],
  )
  def k(dst_hbm, ones_h, z_hbm, out_hbm, didx, ones_v, acc):
    c = lax.axis_index("c")
    s = lax.axis_index("s")
    wid = c * NS + s
    pltpu.sync_copy(ones_h, ones_v)
    pltpu.sync_copy(dst_hbm.at[wid], didx)
    pltpu.sync_copy(z_hbm.at[pl.ds(s * rpt, rpt)], acc.at[pl.ds(s * rpt, rpt)])
    plsc.subcore_barrier()

    @pl.loop(0, nch // G)
    def _(j):
      pltpu.sync_copy(ones_v, acc.at[didx.at[j]], add=True)

    plsc.subcore_barrier()
    pltpu.sync_copy(acc.at[pl.ds(s * rpt, rpt)],
                    out_hbm.at[c, pl.ds(s * rpt, rpt)])

  return k(dst3, ones_hbm, zeros_hbm)


def _sc_scatter(u, src3, dst3, zeros_hbm, npad, nch):
  d = u.shape[1]
  rpt = npad // NS

  @functools.partial(
      pl.kernel,
      mesh=_sc_mesh(),
      compiler_params=_SC_PARAMS,
      out_type=jax.ShapeDtypeStruct((NC, npad, d), jnp.float32),
      scratch_types=[
          pltpu.VMEM((nch // G, G * CH), jnp.int32),
          pltpu.VMEM((nch // G, G * CH), jnp.int32),
          [pltpu.VMEM((G * CH, d), jnp.float32) for _ in range(2)],
          pltpu.VMEM_SHARED((npad, d), jnp.float32),
          [pltpu.SemaphoreType.DMA for _ in range(2)],
      ],
  )
  def k(u_hbm, src_hbm, dst_hbm, z_hbm, out_hbm, sidx, didx, rows, acc, sems):
    c = lax.axis_index("c")
    s = lax.axis_index("s")
    wid = c * NS + s
    nit = nch // G
    pltpu.sync_copy(src_hbm.at[wid], sidx)
    pltpu.sync_copy(dst_hbm.at[wid], didx)
    pltpu.sync_copy(z_hbm.at[pl.ds(s * rpt, rpt)], acc.at[pl.ds(s * rpt, rpt)])
    pltpu.async_copy(u_hbm.at[sidx.at[0]], rows[0], sems[0])
    plsc.subcore_barrier()

    @pl.loop(0, nit, step=2)
    def _(i):
      for b in range(2):
        o = 1 - b
        pltpu.make_async_copy(
            u_hbm.at[pl.ds(0, G * CH)], rows[b], sems[b]).wait()

        @pl.when(i + b + 1 < nit)
        def _():
          pltpu.async_copy(u_hbm.at[sidx.at[i + b + 1]], rows[o], sems[o])

        pltpu.sync_copy(rows[b], acc.at[didx.at[i + b]], add=True)

    plsc.subcore_barrier()
    pltpu.sync_copy(acc.at[pl.ds(s * rpt, rpt)],
                    out_hbm.at[c, pl.ds(s * rpt, rpt)])

  return k(u, src3, dst3, zeros_hbm)


def _tc_pass_a(degp, x, w1, n):
  dh = w1.shape[1]

  def body(degp_ref, x_ref, w_ref, u_ref, dinv_ref):
    deg = degp_ref[0] + degp_ref[1]
    dinv = lax.rsqrt(deg[:n, 0:1] + 1.0)
    h = jnp.dot(x_ref[...], w_ref[...], preferred_element_type=jnp.float32)
    u_ref[...] = h * dinv
    dinv_ref[...] = dinv

  return pl.pallas_call(
      body,
      out_shape=(
          jax.ShapeDtypeStruct((n, dh), jnp.float32),
          jax.ShapeDtypeStruct((n, 1), jnp.float32),
      ),
  )(degp, x, w1)


def _tc_pass_b(s1, u1, dinv, b1, w2, n):
  do = w2.shape[1]

  def body(s_ref, u_ref, dinv_ref, b_ref, w_ref, u2_ref):
    agg = (s_ref[0, :n] + s_ref[1, :n] + u_ref[...]) * dinv_ref[...]
    h1 = jnp.maximum(agg + b_ref[...], 0.0)
    h2 = jnp.dot(h1, w_ref[...], preferred_element_type=jnp.float32)
    u2_ref[...] = h2 * dinv_ref[...]

  return pl.pallas_call(
      body,
      out_shape=jax.ShapeDtypeStruct((n, do), jnp.float32),
  )(s1, u1, dinv, b1, w2)


def _tc_pass_c(s2, u2, dinv, b2, n):
  do = u2.shape[1]

  def body(s_ref, u_ref, dinv_ref, b_ref, out_ref):
    agg = (s_ref[0, :n] + s_ref[1, :n] + u_ref[...]) * dinv_ref[...]
    out_ref[...] = agg + b_ref[...]

  return pl.pallas_call(
      body,
      out_shape=jax.ShapeDtypeStruct((n, do), jnp.float32),
  )(s2, u2, dinv, b2)


def kernel(x, edge_index, W1, b1, W2, b2):
  n = x.shape[0]
  e = edge_index.shape[1]
  dh = W1.shape[1]
  do = W2.shape[1]

  nch = -(-e // (NW * CH * NB)) * NB
  epad = NW * nch * CH
  src = jnp.concatenate(
      [edge_index[0], jnp.zeros((epad - e,), edge_index.dtype)])
  dst = jnp.concatenate(
      [edge_index[1], jnp.full((epad - e,), n, edge_index.dtype)])
  dst3 = dst.reshape(NW, nch // G, G * CH)
  src3 = src.reshape(NW, nch // G, G * CH)

  npad = -(-(n + 1) // (NS * 8)) * (NS * 8)
  ones16 = jnp.ones((G * CH, 16), jnp.float32)
  z16 = jnp.zeros((npad, 16), jnp.float32)
  zh = jnp.zeros((npad, dh), jnp.float32)
  zo = jnp.zeros((npad, do), jnp.float32)

  degp = _sc_degree(dst3, ones16, z16, npad, nch)
  u1, dinv = _tc_pass_a(degp, x, W1, n)
  s1 = _sc_scatter(u1, src3, dst3, zh, npad, nch)
  u2 = _tc_pass_b(s1, u1, dinv, b1, W2, n)
  s2 = _sc_scatter(u2, src3, dst3, zo, npad, nch)
  return _tc_pass_c(s2, u2, dinv, b2, n)
